# baseline (speedup 1.0000x reference)
import functools

import jax
import jax.numpy as jnp
from jax import lax
from jax.experimental import pallas as pl
from jax.experimental.pallas import tpu as pltpu
from jax.experimental.pallas import tpu_sc as plsc

_N = 320000
_D = 128
_S = 1024
_NW = 32
_HW = 16
_C = _N // _NW
_R = 256
_G = 16
_NG = _R // _G
_NCH = 40
_TAIL = _C - _R
_K = 16
_NEG = float("-inf")
_OUTW = 2 * _D


def _make_sc_body(hoff, tail_on):
  def _sc_body(x_hbm, ids_hbm, sc_hbm, meta_hbm,
             res_hbm, pvec_hbm, pid_hbm,
             xb0, xb1, idb0, idb1, sb0, sb1, mvm, stage, pstage, sidb,
             sx0, sx1, si0, si1, ss0, ss1, esem):
    wid = lax.axis_index("s") + lax.axis_index("c")
    base = hoff + wid * _C

    pltpu.sync_copy(meta_hbm.at[pl.ds(wid * 16, 16)], mvm)
    mv = mvm[...]
    prev_id = mv[0]
    next_id = mv[1]

    zero8 = tuple(jnp.zeros((16,), jnp.float32) for _ in range(8))
    ninf8 = tuple(jnp.full((16,), _NEG, jnp.float32) for _ in range(8))

    sidb[...] = jnp.full((16,), -1, jnp.int32)
    pltpu.sync_copy(sidb, pid_hbm.at[pl.ds((2 * wid) * 16, 16)])
    pltpu.sync_copy(sidb, pid_hbm.at[pl.ds((2 * wid + 1) * 16, 16)])

    def emit_row(gc, seg, sums, maxs):
        off = gc * _OUTW
        for j in range(8):
            stage[pl.ds(off + 16 * j, 16)] = sums[j]
            stage[pl.ds(off + _D + 16 * j, 16)] = maxs[j]
        pltpu.async_copy(stage.at[pl.ds(off, _OUTW)],
                         res_hbm.at[pl.ds(seg * _OUTW, _OUTW)], esem)
        gcn = gc + 1

        def drain(_):
            pltpu.make_async_copy(stage, res_hbm.at[pl.ds(0, _K * _OUTW)],
                                  esem).wait()
            return jnp.int32(0)

        return lax.cond(gcn == _K, drain, lambda g: g, gcn)

    def part_emit(slot, seg, sums, maxs, gc):
        for j in range(8):
            pstage[pl.ds(16 * j, 16)] = sums[j]
            pstage[pl.ds(_D + 16 * j, 16)] = maxs[j]
        pltpu.sync_copy(pstage,
                        pvec_hbm.at[pl.ds((2 * wid + slot) * _OUTW, _OUTW)])
        sidb[...] = lax.broadcast(seg, (16,))
        pltpu.sync_copy(sidb, pid_hbm.at[pl.ds((2 * wid + slot) * 16, 16)])
        return gc

    def flush_to(rid, c):
        cur, gc = c[0], c[1]
        sums, maxs = c[2:10], c[10:18]
        started = cur >= 0

        def emit_cur(g):
            return lax.cond(
                cur == prev_id,
                lambda gg: part_emit(0, cur, sums, maxs, gg),
                lambda gg: emit_row(gg, cur, sums, maxs),
                g)

        gc = lax.cond(started, emit_cur, lambda g: g, gc)
        gap_lo = jnp.where(started, cur, prev_id)
        gc = lax.fori_loop(gap_lo + 1, rid,
                           lambda e, g: emit_row(g, e, zero8, ninf8), gc)
        return (rid, gc) + zero8 + ninf8

    bcast_dn = lax.GatherDimensionNumbers(
        offset_dims=(), collapsed_slice_dims=(0,), start_index_map=(0,))

    def accum_row(xb, sgv, j, r, c):
        cur, gc = c[0], c[1]
        sums, maxs = c[2:10], c[10:18]
        xo = r * _D
        xv = [xb[pl.ds(xo + 16 * k, 16)] for k in range(8)]
        sig = lax.gather(sgv, jnp.full((16, 1), j, jnp.int32), bcast_dn,
                         (1,), mode=lax.GatherScatterMode.PROMISE_IN_BOUNDS)
        new_sums = tuple(sums[k] + sig * xv[k] for k in range(8))
        new_maxs = tuple(jnp.maximum(maxs[k], xv[k]) for k in range(8))
        return (cur, gc) + new_sums + new_maxs

    def make_group_body(xb, idb, sb):
        def group_body(q, c):
            idv = idb[pl.ds(q * _G, _G)]
            sgv = sb[pl.ds(q * _G, _G)]

            def fast(cc):
                for j in range(_G):
                    cc = accum_row(xb, sgv, j, q * _G + j, cc)
                return cc

            lanes = lax.broadcasted_iota(jnp.int32, (_G,), 0)

            def slow(cc):
                def srow(j, ccc):
                    rid = jnp.sum(jnp.where(lanes == j, idv, 0))
                    ccc = lax.cond(rid != ccc[0],
                                   lambda t: flush_to(rid, t),
                                   lambda t: t, ccc)
                    return accum_row(xb, sgv, j, q * _G + j, ccc)
                return lax.fori_loop(0, _G, srow, cc)

            return lax.cond(idv[_G - 1] == c[0], fast, slow, c)
        return group_body

    def start_chunk(row_off, xb, idb, sb, sx, si, ss):
        pltpu.async_copy(x_hbm.at[pl.ds((base + row_off) * _D, _R * _D)],
                         xb, sx)
        pltpu.async_copy(ids_hbm.at[pl.ds(base + row_off, _R)], idb, si)
        pltpu.async_copy(sc_hbm.at[pl.ds(base + row_off, _R)], sb, ss)

    start_chunk(0, xb0, idb0, sb0, sx0, si0, ss0)
    start_chunk(_R, xb1, idb1, sb1, sx1, si1, ss1)

    def do_stage(g, xb, idb, sb, sx, si, ss, prefetch, q_lo, carry):
        pltpu.make_async_copy(x_hbm.at[pl.ds(0, _R * _D)], xb, sx).wait()
        pltpu.make_async_copy(ids_hbm.at[pl.ds(0, _R)], idb, si).wait()
        pltpu.make_async_copy(sc_hbm.at[pl.ds(0, _R)], sb, ss).wait()
        carry = lax.fori_loop(q_lo, _NG, make_group_body(xb, idb, sb), carry)

        @pl.when(prefetch)
        def _():
            row_off = jnp.minimum((g + 2) * _R, _TAIL)
            start_chunk(row_off, xb, idb, sb, sx, si, ss)

        return carry

    def outer(i, carry):
        carry = do_stage(2 * i, xb0, idb0, sb0, sx0, si0, ss0,
                         i >= 0, 0, carry)
        carry = do_stage(2 * i + 1, xb1, idb1, sb1, sx1, si1, ss1,
                         i >= 0, 0, carry)
        return carry

    carry0 = (jnp.int32(-2), jnp.int32(0)) + zero8 + ninf8
    carry = lax.fori_loop(0, (_NCH - 2) // 2, outer, carry0)
    carry = do_stage(_NCH - 2, xb0, idb0, sb0, sx0, si0, ss0,
                     jnp.bool_(False), 0, carry)
    carry = do_stage(_NCH - 1, xb1, idb1, sb1, sx1, si1, ss1,
                     jnp.bool_(False), _NG - 1, carry)

    cur, gc = carry[0], carry[1]
    sums, maxs = carry[2:10], carry[10:18]

    def fin(g):
        return lax.cond(
            cur == next_id,
            lambda gg: part_emit(1, cur, sums, maxs, gg),
            lambda gg: lax.cond(
                cur == prev_id,
                lambda g3: part_emit(0, cur, sums, maxs, g3),
                lambda g3: emit_row(g3, cur, sums, maxs),
                gg),
            g)

    gc = fin(gc)
    if tail_on:
        gc = lax.cond(wid == _HW - 1,
                      lambda g: lax.fori_loop(cur + 1, _S,
                                              lambda e, gg: emit_row(gg, e, zero8, ninf8), g),
                      lambda g: g, gc)

    def drain_one(_, u):
        pltpu.make_async_copy(stage.at[pl.ds(0, _OUTW)],
                              res_hbm.at[pl.ds(0, _OUTW)], esem).wait()
        return u

    lax.fori_loop(0, gc, drain_one, jnp.int32(0))
  return _sc_body


_SB = 16000


def _score_body(x_ref, w_ref, b_ref, o_ref):
    z = jnp.sum(x_ref[...] * w_ref[...], axis=1) + b_ref[0]
    o_ref[...] = (1.0 / (1.0 + jnp.exp(-z))).reshape(8, _SB // 8)


def _fixup_body(s0_ref, s1_ref, pv_ref, pid_ref, hi_ref, out_ref):
    ids = pid_ref[...][:, 0:1]
    seg = lax.broadcasted_iota(jnp.int32, (2 * _NW, _S), 1)
    m = ids == seg
    mf = m.astype(jnp.float32)
    psum = pv_ref[...][:, :_D]
    comb_sum = lax.dot_general(mf, psum, (((0,), (0,)), ((), ())),
                               preferred_element_type=jnp.float32)
    seg_col = lax.broadcasted_iota(jnp.int32, (_S, 1), 0)

    comb_max = jnp.full((_S, _D), _NEG, jnp.float32)
    shared = jnp.zeros((_S, 1), jnp.bool_)
    ids_all = pid_ref[...]
    for p in range(2 * _NW):
        idp = ids_all[p, 0]
        row = pv_ref[p:p + 1, _D:]
        col = seg_col == idp
        comb_max = jnp.maximum(comb_max, jnp.where(col, row, _NEG))
        shared = jnp.logical_or(shared, col)
    merged = jnp.concatenate([comb_sum, comb_max], axis=1)
    base_rows = jnp.where(seg_col <= hi_ref[0], s0_ref[...], s1_ref[...])
    out_ref[...] = jnp.where(shared, merged, base_rows)


@functools.partial(jax.jit)
def kernel(x, batch, W, b):
    batch = batch.astype(jnp.int32)
    score = pl.pallas_call(
        _score_body,
        grid=(_N // _SB,),
        in_specs=[
            pl.BlockSpec((_SB, _D), lambda i: (i, 0)),
            pl.BlockSpec((1, _D), lambda i: (0, 0)),
            pl.BlockSpec(memory_space=pltpu.SMEM),
        ],
        out_specs=pl.BlockSpec((8, _SB // 8), lambda i: (i, 0)),
        out_shape=jax.ShapeDtypeStruct((_N // _SB * 8, _SB // 8),
                                       jnp.float32),
    )(x, W.astype(jnp.float32), b.astype(jnp.float32))
    prevs = jnp.concatenate(
        [jnp.full((1,), -1, jnp.int32), batch[_C - 1::_C][: _NW - 1]])
    nexts = jnp.concatenate(
        [batch[_C::_C][: _NW - 1], jnp.full((1,), _S, jnp.int32)])
    meta = jnp.pad(jnp.stack([prevs, nexts], axis=1), ((0, 0), (0, 14)))

    mesh = plsc.VectorSubcoreMesh(core_axis_name="c", subcore_axis_name="s",
                                  num_cores=1)
    scratch_types = [
        pltpu.VMEM((_R * _D,), jnp.float32),
        pltpu.VMEM((_R * _D,), jnp.float32),
        pltpu.VMEM((_R,), jnp.int32),
        pltpu.VMEM((_R,), jnp.int32),
        pltpu.VMEM((_R,), jnp.float32),
        pltpu.VMEM((_R,), jnp.float32),
        pltpu.VMEM((16,), jnp.int32),
        pltpu.VMEM((_K * _OUTW,), jnp.float32),
        pltpu.VMEM((_OUTW,), jnp.float32),
        pltpu.VMEM((16,), jnp.int32),
    ] + [pltpu.SemaphoreType.DMA] * 7
    out_type = (
        jax.ShapeDtypeStruct((_S * _OUTW,), jnp.float32),
        jax.ShapeDtypeStruct((2 * _HW * _OUTW,), jnp.float32),
        jax.ShapeDtypeStruct((2 * _HW * 16,), jnp.int32),
    )
    xf = x.reshape(_N * _D)
    sf = score.reshape(_N)
    halves = []
    for k in (0, 1):
        meta_k = meta[k * _HW:(k + 1) * _HW].reshape(_HW * 16)
        halves.append(pl.kernel(
            _make_sc_body(k * (_N // 2), k == 1),
            out_type=out_type,
            mesh=mesh,
            compiler_params=pltpu.CompilerParams(
                needs_layout_passes=False),
            scratch_types=scratch_types,
        )(xf, batch, sf, meta_k))
    (scr0, pv0, pid0), (scr1, pv1, pid1) = halves
    pvec = jnp.concatenate([pv0, pv1]).reshape(2 * _NW, _OUTW)
    pid = jnp.concatenate([pid0, pid1]).reshape(2 * _NW, 16)
    hi_split = batch[_N // 2 - 1:_N // 2]

    return pl.pallas_call(
        _fixup_body,
        in_specs=[
            pl.BlockSpec((_S, _OUTW), lambda: (0, 0)),
            pl.BlockSpec((_S, _OUTW), lambda: (0, 0)),
            pl.BlockSpec((2 * _NW, _OUTW), lambda: (0, 0)),
            pl.BlockSpec((2 * _NW, 16), lambda: (0, 0)),
            pl.BlockSpec(memory_space=pltpu.SMEM),
        ],
        out_shape=jax.ShapeDtypeStruct((_S, _OUTW), jnp.float32),
    )(scr0.reshape(_S, _OUTW), scr1.reshape(_S, _OUTW), pvec, pid,
      hi_split)


# speedup vs baseline: 1.4709x; 1.4709x over previous
import functools

import jax
import jax.numpy as jnp
from jax import lax
from jax.experimental import pallas as pl
from jax.experimental.pallas import tpu as pltpu
from jax.experimental.pallas import tpu_sc as plsc

_N = 320000
_D = 128
_S = 1024
_NW = 32
_C = _N // _NW
_R = 256
_G = 16
_NG = _R // _G
_NCH = 40
_TAIL = _C - _R
_K = 16
_NEG = float("-inf")
_OUTW = 2 * _D


def _sc_body(x_hbm, ids_hbm, sc_hbm, meta_hbm,
             res_hbm, pvec_hbm, pid_hbm,
             xb0, xb1, idb0, idb1, sb0, sb1, mvm, stage, pstage, sidb,
             sx0, sx1, si0, si1, ss0, ss1, esem):
    nc = 2
    wid = lax.axis_index("s") * nc + lax.axis_index("c")
    base = wid * _C

    pltpu.sync_copy(meta_hbm.at[pl.ds(wid * 16, 16)], mvm)
    mv = mvm[...]
    prev_id = mv[0]
    next_id = mv[1]

    zero8 = tuple(jnp.zeros((16,), jnp.float32) for _ in range(8))
    ninf8 = tuple(jnp.full((16,), _NEG, jnp.float32) for _ in range(8))

    sidb[...] = jnp.full((16,), -1, jnp.int32)
    pltpu.sync_copy(sidb, pid_hbm.at[pl.ds((2 * wid) * 16, 16)])
    pltpu.sync_copy(sidb, pid_hbm.at[pl.ds((2 * wid + 1) * 16, 16)])

    def emit_row(gc, seg, sums, maxs):
        off = gc * _OUTW
        for j in range(8):
            stage[pl.ds(off + 16 * j, 16)] = sums[j]
            stage[pl.ds(off + _D + 16 * j, 16)] = maxs[j]
        pltpu.async_copy(stage.at[pl.ds(off, _OUTW)],
                         res_hbm.at[pl.ds(seg * _OUTW, _OUTW)], esem)
        gcn = gc + 1

        def drain(_):
            pltpu.make_async_copy(stage, res_hbm.at[pl.ds(0, _K * _OUTW)],
                                  esem).wait()
            return jnp.int32(0)

        return lax.cond(gcn == _K, drain, lambda g: g, gcn)

    def part_emit(slot, seg, sums, maxs, gc):
        for j in range(8):
            pstage[pl.ds(16 * j, 16)] = sums[j]
            pstage[pl.ds(_D + 16 * j, 16)] = maxs[j]
        pltpu.sync_copy(pstage,
                        pvec_hbm.at[pl.ds((2 * wid + slot) * _OUTW, _OUTW)])
        sidb[...] = lax.broadcast(seg, (16,))
        pltpu.sync_copy(sidb, pid_hbm.at[pl.ds((2 * wid + slot) * 16, 16)])
        return gc

    def flush_to(rid, c):
        cur, gc = c[0], c[1]
        sums, maxs = c[2:10], c[10:18]
        started = cur >= 0

        def emit_cur(g):
            return lax.cond(
                cur == prev_id,
                lambda gg: part_emit(0, cur, sums, maxs, gg),
                lambda gg: emit_row(gg, cur, sums, maxs),
                g)

        gc = lax.cond(started, emit_cur, lambda g: g, gc)
        gap_lo = jnp.where(started, cur, prev_id)
        gc = lax.fori_loop(gap_lo + 1, rid,
                           lambda e, g: emit_row(g, e, zero8, ninf8), gc)
        return (rid, gc) + zero8 + ninf8

    bcast_dn = lax.GatherDimensionNumbers(
        offset_dims=(), collapsed_slice_dims=(0,), start_index_map=(0,))

    def accum_row(xb, sgv, j, r, c):
        cur, gc = c[0], c[1]
        sums, maxs = c[2:10], c[10:18]
        xo = r * _D
        xv = [xb[pl.ds(xo + 16 * k, 16)] for k in range(8)]
        sig = lax.gather(sgv, jnp.full((16, 1), j, jnp.int32), bcast_dn,
                         (1,), mode=lax.GatherScatterMode.PROMISE_IN_BOUNDS)
        new_sums = tuple(sums[k] + sig * xv[k] for k in range(8))
        new_maxs = tuple(jnp.maximum(maxs[k], xv[k]) for k in range(8))
        return (cur, gc) + new_sums + new_maxs

    def make_group_body(xb, idb, sb):
        def group_body(q, c):
            idv = idb[pl.ds(q * _G, _G)]
            sgv = sb[pl.ds(q * _G, _G)]

            def fast(cc):
                for j in range(_G):
                    cc = accum_row(xb, sgv, j, q * _G + j, cc)
                return cc

            lanes = lax.broadcasted_iota(jnp.int32, (_G,), 0)

            def slow(cc):
                def srow(j, ccc):
                    rid = jnp.sum(jnp.where(lanes == j, idv, 0))
                    ccc = lax.cond(rid != ccc[0],
                                   lambda t: flush_to(rid, t),
                                   lambda t: t, ccc)
                    return accum_row(xb, sgv, j, q * _G + j, ccc)
                return lax.fori_loop(0, _G, srow, cc)

            return lax.cond(idv[_G - 1] == c[0], fast, slow, c)
        return group_body

    def drain_one(_, u):
        pltpu.make_async_copy(stage.at[pl.ds(0, _OUTW)],
                              res_hbm.at[pl.ds(0, _OUTW)], esem).wait()
        return u

    def start_chunk(row_off, xb, idb, sb, sx, si, ss):
        pltpu.async_copy(x_hbm.at[pl.ds((base + row_off) * _D, _R * _D)],
                         xb, sx)
        pltpu.async_copy(ids_hbm.at[pl.ds(base + row_off, _R)], idb, si)
        pltpu.async_copy(sc_hbm.at[pl.ds(base + row_off, _R)], sb, ss)

    start_chunk(0, xb0, idb0, sb0, sx0, si0, ss0)
    start_chunk(_R, xb1, idb1, sb1, sx1, si1, ss1)

    def do_stage(g, xb, idb, sb, sx, si, ss, prefetch, q_lo, carry):
        pltpu.make_async_copy(x_hbm.at[pl.ds(0, _R * _D)], xb, sx).wait()
        pltpu.make_async_copy(ids_hbm.at[pl.ds(0, _R)], idb, si).wait()
        pltpu.make_async_copy(sc_hbm.at[pl.ds(0, _R)], sb, ss).wait()
        carry = lax.fori_loop(q_lo, _NG, make_group_body(xb, idb, sb), carry)
        carry = (carry[0], lax.fori_loop(0, carry[1], drain_one,
                                         carry[1]) * 0) + carry[2:]

        @pl.when(prefetch)
        def _():
            row_off = jnp.minimum((g + 2) * _R, _TAIL)
            start_chunk(row_off, xb, idb, sb, sx, si, ss)

        return carry

    def outer(i, carry):
        carry = do_stage(2 * i, xb0, idb0, sb0, sx0, si0, ss0,
                         i >= 0, 0, carry)
        carry = do_stage(2 * i + 1, xb1, idb1, sb1, sx1, si1, ss1,
                         i >= 0, 0, carry)
        return carry

    carry0 = (jnp.int32(-2), jnp.int32(0)) + zero8 + ninf8
    carry = lax.fori_loop(0, (_NCH - 2) // 2, outer, carry0)
    carry = do_stage(_NCH - 2, xb0, idb0, sb0, sx0, si0, ss0,
                     jnp.bool_(False), 0, carry)
    carry = do_stage(_NCH - 1, xb1, idb1, sb1, sx1, si1, ss1,
                     jnp.bool_(False), _NG - 1, carry)

    cur, gc = carry[0], carry[1]
    sums, maxs = carry[2:10], carry[10:18]

    def fin(g):
        return lax.cond(
            cur == next_id,
            lambda gg: part_emit(1, cur, sums, maxs, gg),
            lambda gg: lax.cond(
                cur == prev_id,
                lambda g3: part_emit(0, cur, sums, maxs, g3),
                lambda g3: emit_row(g3, cur, sums, maxs),
                gg),
            g)

    gc = fin(gc)
    gc = lax.cond(wid == _NW - 1,
                  lambda g: lax.fori_loop(cur + 1, _S,
                                          lambda e, gg: emit_row(gg, e, zero8, ninf8), g),
                  lambda g: g, gc)

    lax.fori_loop(0, gc, drain_one, jnp.int32(0))


_SB = 16000


def _score_body(x_ref, w_ref, b_ref, o_ref):
    z = jnp.sum(x_ref[...] * w_ref[...], axis=1) + b_ref[0]
    o_ref[...] = (1.0 / (1.0 + jnp.exp(-z))).reshape(8, _SB // 8)


def _fixup_body(scr_ref, pv_ref, pid_ref, out_ref):
    ids = pid_ref[...][:, 0:1]
    seg = lax.broadcasted_iota(jnp.int32, (2 * _NW, _S), 1)
    m = ids == seg
    mf = m.astype(jnp.float32)
    psum = pv_ref[...][:, :_D]
    comb_sum = lax.dot_general(mf, psum, (((0,), (0,)), ((), ())),
                               preferred_element_type=jnp.float32)
    seg_col = lax.broadcasted_iota(jnp.int32, (_S, 1), 0)

    comb_max = jnp.full((_S, _D), _NEG, jnp.float32)
    shared = jnp.zeros((_S, 1), jnp.bool_)
    ids_all = pid_ref[...]
    for p in range(2 * _NW):
        idp = ids_all[p, 0]
        row = pv_ref[p:p + 1, _D:]
        col = seg_col == idp
        comb_max = jnp.maximum(comb_max, jnp.where(col, row, _NEG))
        shared = jnp.logical_or(shared, col)
    merged = jnp.concatenate([comb_sum, comb_max], axis=1)
    out_ref[...] = jnp.where(shared, merged, scr_ref[...])


@functools.partial(jax.jit)
def kernel(x, batch, W, b):
    batch = batch.astype(jnp.int32)
    score = pl.pallas_call(
        _score_body,
        grid=(_N // _SB,),
        in_specs=[
            pl.BlockSpec((_SB, _D), lambda i: (i, 0)),
            pl.BlockSpec((1, _D), lambda i: (0, 0)),
            pl.BlockSpec(memory_space=pltpu.SMEM),
        ],
        out_specs=pl.BlockSpec((8, _SB // 8), lambda i: (i, 0)),
        out_shape=jax.ShapeDtypeStruct((_N // _SB * 8, _SB // 8),
                                       jnp.float32),
    )(x, W.astype(jnp.float32), b.astype(jnp.float32))
    prevs = jnp.concatenate(
        [jnp.full((1,), -1, jnp.int32), batch[_C - 1::_C][: _NW - 1]])
    nexts = jnp.concatenate(
        [batch[_C::_C][: _NW - 1], jnp.full((1,), _S, jnp.int32)])
    meta = jnp.pad(jnp.stack([prevs, nexts], axis=1), ((0, 0), (0, 14)))

    mesh = plsc.VectorSubcoreMesh(core_axis_name="c", subcore_axis_name="s")
    scratch, pvec, pid = pl.kernel(
        _sc_body,
        out_type=(
            jax.ShapeDtypeStruct((_S * _OUTW,), jnp.float32),
            jax.ShapeDtypeStruct((2 * _NW * _OUTW,), jnp.float32),
            jax.ShapeDtypeStruct((2 * _NW * 16,), jnp.int32),
        ),
        mesh=mesh,
        compiler_params=pltpu.CompilerParams(needs_layout_passes=False),
        scratch_types=[
            pltpu.VMEM((_R * _D,), jnp.float32),
            pltpu.VMEM((_R * _D,), jnp.float32),
            pltpu.VMEM((_R,), jnp.int32),
            pltpu.VMEM((_R,), jnp.int32),
            pltpu.VMEM((_R,), jnp.float32),
            pltpu.VMEM((_R,), jnp.float32),
            pltpu.VMEM((16,), jnp.int32),
            pltpu.VMEM((_K * _OUTW,), jnp.float32),
            pltpu.VMEM((_OUTW,), jnp.float32),
            pltpu.VMEM((16,), jnp.int32),
            pltpu.SemaphoreType.DMA,
            pltpu.SemaphoreType.DMA,
            pltpu.SemaphoreType.DMA,
            pltpu.SemaphoreType.DMA,
            pltpu.SemaphoreType.DMA,
            pltpu.SemaphoreType.DMA,
            pltpu.SemaphoreType.DMA,
        ],
    )(x.reshape(_N * _D), batch, score.reshape(_N), meta.reshape(_NW * 16))

    return pl.pallas_call(
        _fixup_body,
        out_shape=jax.ShapeDtypeStruct((_S, _OUTW), jnp.float32),
    )(scratch.reshape(_S, _OUTW), pvec.reshape(2 * _NW, _OUTW),
      pid.reshape(2 * _NW, 16))


# speedup vs baseline: 1.4753x; 1.0029x over previous
import functools

import jax
import jax.numpy as jnp
from jax import lax
from jax.experimental import pallas as pl
from jax.experimental.pallas import tpu as pltpu
from jax.experimental.pallas import tpu_sc as plsc

_N = 320000
_D = 128
_S = 1024
_NW = 32
_C = _N // _NW
_R = 256
_G = 16
_NG = _R // _G
_NCH = 40
_TAIL = _C - _R
_K = 16
_NEG = float("-inf")
_OUTW = 2 * _D


def _sc_body(x_hbm, ids_hbm, sc_hbm, meta_hbm,
             res_hbm, pvec_hbm, pid_hbm,
             xb0, xb1, idb0, idb1, sb0, sb1, mvm, stage, pstage, sidb,
             sx0, sx1, si0, si1, ss0, ss1, esem):
    nc = 2
    wid = lax.axis_index("s") * nc + lax.axis_index("c")
    base = wid * _C

    pltpu.sync_copy(meta_hbm.at[pl.ds(wid * 16, 16)], mvm)
    mv = mvm[...]
    prev_id = mv[0]
    next_id = mv[1]

    zero8 = tuple(jnp.zeros((16,), jnp.float32) for _ in range(8))
    ninf8 = tuple(jnp.full((16,), _NEG, jnp.float32) for _ in range(8))

    sidb[...] = jnp.full((16,), -1, jnp.int32)
    pltpu.sync_copy(sidb, pid_hbm.at[pl.ds((2 * wid) * 16, 16)])
    pltpu.sync_copy(sidb, pid_hbm.at[pl.ds((2 * wid + 1) * 16, 16)])

    def emit_row(gc, seg, sums, maxs):
        off = gc * _OUTW
        for j in range(8):
            stage[pl.ds(off + 16 * j, 16)] = sums[j]
            stage[pl.ds(off + _D + 16 * j, 16)] = maxs[j]
        pltpu.async_copy(stage.at[pl.ds(off, _OUTW)],
                         res_hbm.at[pl.ds(seg * _OUTW, _OUTW)], esem)
        gcn = gc + 1

        def drain(_):
            pltpu.make_async_copy(stage, res_hbm.at[pl.ds(0, _K * _OUTW)],
                                  esem).wait()
            return jnp.int32(0)

        return lax.cond(gcn == _K, drain, lambda g: g, gcn)

    def part_emit(slot, seg, sums, maxs, gc):
        for j in range(8):
            pstage[pl.ds(16 * j, 16)] = sums[j]
            pstage[pl.ds(_D + 16 * j, 16)] = maxs[j]
        pltpu.sync_copy(pstage,
                        pvec_hbm.at[pl.ds((2 * wid + slot) * _OUTW, _OUTW)])
        sidb[...] = lax.broadcast(seg, (16,))
        pltpu.sync_copy(sidb, pid_hbm.at[pl.ds((2 * wid + slot) * 16, 16)])
        return gc

    def flush_to(rid, c):
        cur, gc = c[0], c[1]
        sums, maxs = c[2:10], c[10:18]
        started = cur >= 0

        def emit_cur(g):
            return lax.cond(
                cur == prev_id,
                lambda gg: part_emit(0, cur, sums, maxs, gg),
                lambda gg: emit_row(gg, cur, sums, maxs),
                g)

        gc = lax.cond(started, emit_cur, lambda g: g, gc)
        gap_lo = jnp.where(started, cur, prev_id)
        gc = lax.fori_loop(gap_lo + 1, rid,
                           lambda e, g: emit_row(g, e, zero8, ninf8), gc)
        return (rid, gc) + zero8 + ninf8

    bcast_dn = lax.GatherDimensionNumbers(
        offset_dims=(), collapsed_slice_dims=(0,), start_index_map=(0,))

    def accum_row(xb, sgv, j, r, c):
        cur, gc = c[0], c[1]
        sums, maxs = c[2:10], c[10:18]
        xo = r * _D
        xv = [xb[pl.ds(xo + 16 * k, 16)] for k in range(8)]
        sig = lax.gather(sgv, jnp.full((16, 1), j, jnp.int32), bcast_dn,
                         (1,), mode=lax.GatherScatterMode.PROMISE_IN_BOUNDS)
        new_sums = tuple(sums[k] + sig * xv[k] for k in range(8))
        new_maxs = tuple(jnp.maximum(maxs[k], xv[k]) for k in range(8))
        return (cur, gc) + new_sums + new_maxs

    def make_group_body(xb, idb, sb):
        lanes = lax.broadcasted_iota(jnp.int32, (_G,), 0)

        def group16(q, cc):
            idv = idb[pl.ds(q * _G, _G)]
            sgv = sb[pl.ds(q * _G, _G)]

            def fast(t):
                for j in range(_G):
                    t = accum_row(xb, sgv, j, q * _G + j, t)
                return t

            def slow(t):
                def srow(j, tt):
                    rid = jnp.sum(jnp.where(lanes == j, idv, 0))
                    tt = lax.cond(rid != tt[0],
                                  lambda u: flush_to(rid, u),
                                  lambda u: u, tt)
                    return accum_row(xb, sgv, j, q * _G + j, tt)
                return lax.fori_loop(0, _G, srow, t)

            return lax.cond(idv[_G - 1] == cc[0], fast, slow, cc)

        def group_body(q2, c):
            qa = 2 * q2
            qb = qa + 1
            idvb = idb[pl.ds(qb * _G, _G)]

            def fast32(cc):
                sgva = sb[pl.ds(qa * _G, _G)]
                sgvb = sb[pl.ds(qb * _G, _G)]
                for j in range(_G):
                    cc = accum_row(xb, sgva, j, qa * _G + j, cc)
                for j in range(_G):
                    cc = accum_row(xb, sgvb, j, qb * _G + j, cc)
                return cc

            def slow32(cc):
                return group16(qb, group16(qa, cc))

            return lax.cond(idvb[_G - 1] == c[0], fast32, slow32, c)
        return group_body, group16

    def drain_one(_, u):
        pltpu.make_async_copy(stage.at[pl.ds(0, _OUTW)],
                              res_hbm.at[pl.ds(0, _OUTW)], esem).wait()
        return u

    def start_chunk(row_off, xb, idb, sb, sx, si, ss):
        pltpu.async_copy(x_hbm.at[pl.ds((base + row_off) * _D, _R * _D)],
                         xb, sx)
        pltpu.async_copy(ids_hbm.at[pl.ds(base + row_off, _R)], idb, si)
        pltpu.async_copy(sc_hbm.at[pl.ds(base + row_off, _R)], sb, ss)

    start_chunk(0, xb0, idb0, sb0, sx0, si0, ss0)
    start_chunk(_R, xb1, idb1, sb1, sx1, si1, ss1)

    def do_stage(g, xb, idb, sb, sx, si, ss, prefetch, tail, carry):
        pltpu.make_async_copy(x_hbm.at[pl.ds(0, _R * _D)], xb, sx).wait()
        pltpu.make_async_copy(ids_hbm.at[pl.ds(0, _R)], idb, si).wait()
        pltpu.make_async_copy(sc_hbm.at[pl.ds(0, _R)], sb, ss).wait()
        group_body, group16 = make_group_body(xb, idb, sb)
        if tail:
            carry = group16(_NG - 1, carry)
        else:
            carry = lax.fori_loop(0, _NG // 2, group_body, carry)
        carry = (carry[0], lax.fori_loop(0, carry[1], drain_one,
                                         carry[1]) * 0) + carry[2:]

        @pl.when(prefetch)
        def _():
            row_off = jnp.minimum((g + 2) * _R, _TAIL)
            start_chunk(row_off, xb, idb, sb, sx, si, ss)

        return carry

    def outer(i, carry):
        carry = do_stage(2 * i, xb0, idb0, sb0, sx0, si0, ss0,
                         i >= 0, False, carry)
        carry = do_stage(2 * i + 1, xb1, idb1, sb1, sx1, si1, ss1,
                         i >= 0, False, carry)
        return carry

    carry0 = (jnp.int32(-2), jnp.int32(0)) + zero8 + ninf8
    carry = lax.fori_loop(0, (_NCH - 2) // 2, outer, carry0)
    carry = do_stage(_NCH - 2, xb0, idb0, sb0, sx0, si0, ss0,
                     jnp.bool_(False), False, carry)
    carry = do_stage(_NCH - 1, xb1, idb1, sb1, sx1, si1, ss1,
                     jnp.bool_(False), True, carry)

    cur, gc = carry[0], carry[1]
    sums, maxs = carry[2:10], carry[10:18]

    def fin(g):
        return lax.cond(
            cur == next_id,
            lambda gg: part_emit(1, cur, sums, maxs, gg),
            lambda gg: lax.cond(
                cur == prev_id,
                lambda g3: part_emit(0, cur, sums, maxs, g3),
                lambda g3: emit_row(g3, cur, sums, maxs),
                gg),
            g)

    gc = fin(gc)
    gc = lax.cond(wid == _NW - 1,
                  lambda g: lax.fori_loop(cur + 1, _S,
                                          lambda e, gg: emit_row(gg, e, zero8, ninf8), g),
                  lambda g: g, gc)

    lax.fori_loop(0, gc, drain_one, jnp.int32(0))


_SB = 16000


def _score_body(x_ref, w_ref, b_ref, o_ref):
    z = jnp.sum(x_ref[...] * w_ref[...], axis=1) + b_ref[0]
    o_ref[...] = (1.0 / (1.0 + jnp.exp(-z))).reshape(8, _SB // 8)


def _fixup_body(scr_ref, pv_ref, pid_ref, out_ref):
    ids = pid_ref[...][:, 0:1]
    seg = lax.broadcasted_iota(jnp.int32, (2 * _NW, _S), 1)
    m = ids == seg
    mf = m.astype(jnp.float32)
    psum = pv_ref[...][:, :_D]
    comb_sum = lax.dot_general(mf, psum, (((0,), (0,)), ((), ())),
                               preferred_element_type=jnp.float32)
    seg_col = lax.broadcasted_iota(jnp.int32, (_S, 1), 0)

    comb_max = jnp.full((_S, _D), _NEG, jnp.float32)
    shared = jnp.zeros((_S, 1), jnp.bool_)
    ids_all = pid_ref[...]
    for p in range(2 * _NW):
        idp = ids_all[p, 0]
        row = pv_ref[p:p + 1, _D:]
        col = seg_col == idp
        comb_max = jnp.maximum(comb_max, jnp.where(col, row, _NEG))
        shared = jnp.logical_or(shared, col)
    merged = jnp.concatenate([comb_sum, comb_max], axis=1)
    out_ref[...] = jnp.where(shared, merged, scr_ref[...])


@functools.partial(jax.jit)
def kernel(x, batch, W, b):
    batch = batch.astype(jnp.int32)
    score = pl.pallas_call(
        _score_body,
        grid=(_N // _SB,),
        in_specs=[
            pl.BlockSpec((_SB, _D), lambda i: (i, 0)),
            pl.BlockSpec((1, _D), lambda i: (0, 0)),
            pl.BlockSpec(memory_space=pltpu.SMEM),
        ],
        out_specs=pl.BlockSpec((8, _SB // 8), lambda i: (i, 0)),
        out_shape=jax.ShapeDtypeStruct((_N // _SB * 8, _SB // 8),
                                       jnp.float32),
    )(x, W.astype(jnp.float32), b.astype(jnp.float32))
    prevs = jnp.concatenate(
        [jnp.full((1,), -1, jnp.int32), batch[_C - 1::_C][: _NW - 1]])
    nexts = jnp.concatenate(
        [batch[_C::_C][: _NW - 1], jnp.full((1,), _S, jnp.int32)])
    meta = jnp.pad(jnp.stack([prevs, nexts], axis=1), ((0, 0), (0, 14)))

    mesh = plsc.VectorSubcoreMesh(core_axis_name="c", subcore_axis_name="s")
    scratch, pvec, pid = pl.kernel(
        _sc_body,
        out_type=(
            jax.ShapeDtypeStruct((_S * _OUTW,), jnp.float32),
            jax.ShapeDtypeStruct((2 * _NW * _OUTW,), jnp.float32),
            jax.ShapeDtypeStruct((2 * _NW * 16,), jnp.int32),
        ),
        mesh=mesh,
        compiler_params=pltpu.CompilerParams(needs_layout_passes=False),
        scratch_types=[
            pltpu.VMEM((_R * _D,), jnp.float32),
            pltpu.VMEM((_R * _D,), jnp.float32),
            pltpu.VMEM((_R,), jnp.int32),
            pltpu.VMEM((_R,), jnp.int32),
            pltpu.VMEM((_R,), jnp.float32),
            pltpu.VMEM((_R,), jnp.float32),
            pltpu.VMEM((16,), jnp.int32),
            pltpu.VMEM((_K * _OUTW,), jnp.float32),
            pltpu.VMEM((_OUTW,), jnp.float32),
            pltpu.VMEM((16,), jnp.int32),
            pltpu.SemaphoreType.DMA,
            pltpu.SemaphoreType.DMA,
            pltpu.SemaphoreType.DMA,
            pltpu.SemaphoreType.DMA,
            pltpu.SemaphoreType.DMA,
            pltpu.SemaphoreType.DMA,
            pltpu.SemaphoreType.DMA,
        ],
    )(x.reshape(_N * _D), batch, score.reshape(_N), meta.reshape(_NW * 16))

    return pl.pallas_call(
        _fixup_body,
        out_shape=jax.ShapeDtypeStruct((_S, _OUTW), jnp.float32),
    )(scratch.reshape(_S, _OUTW), pvec.reshape(2 * _NW, _OUTW),
      pid.reshape(2 * _NW, 16))
